# baseline (device time: 47098 ns/iter reference)
import jax
import jax.numpy as jnp
from jax import lax
from jax.experimental import pallas as pl
from jax.experimental.pallas import tpu as pltpu

N_DEV = 4


def kernel(x, w_mat):
    m, k_local = x.shape
    _, n = w_mat.shape
    ch = m // N_DEV
    cw = n // 2

    def body(x_ref, w_ref, out_ref, acc_ref, comm_ref, send_sems, recv_sems):
        my = lax.axis_index("i")
        bit0 = lax.rem(my, 2)
        hi = my // 2
        p_a = my + 1 - 2 * bit0
        p_b = 3 - my
        b1 = lax.rem(bit0 + hi, 2)

        barrier_sem = pltpu.get_barrier_semaphore()
        for nbr in (p_a, p_b):
            pl.semaphore_signal(
                barrier_sem, inc=1,
                device_id=(nbr,), device_id_type=pl.DeviceIdType.MESH,
            )
        pl.semaphore_wait(barrier_sem, 2)

        send1 = (2 * (1 - b1), 2 * (1 - hi))
        own = (2 * b1 + hi, 2 * hi + bit0)
        fwd = (2 * b1 + (1 - hi), 2 * hi + (1 - bit0))
        o_first = (1 - hi, bit0)
        o_second = (hi, 1 - bit0)
        part1 = (p_a, p_b)
        part2 = (p_b, p_a)

        def rcopy(src, dst, h, sem_slot, dev):
            rdma = pltpu.make_async_remote_copy(
                src_ref=src, dst_ref=dst,
                send_sem=send_sems.at[h, sem_slot],
                recv_sem=recv_sems.at[h, sem_slot],
                device_id=(dev,), device_id_type=pl.DeviceIdType.MESH,
            )
            rdma.start()
            return rdma

        def gemm_block(h, blk_idx, nblk):
            return jnp.dot(
                x_ref[pl.ds(blk_idx * ch, nblk * ch), :],
                w_ref[:, h * cw:(h + 1) * cw],
                preferred_element_type=jnp.float32,
            ).reshape(nblk, ch, cw)

        rd1 = [[None, None], [None, None]]
        for h in range(2):
            i = send1[h] + o_first[h]
            acc_ref[h, i] = gemm_block(h, i, 1)[0]
            rd1[h][0] = rcopy(acc_ref.at[h, i], comm_ref.at[h, 0],
                              h, 0, part1[h])
        for h in range(2):
            i = send1[h] + o_second[h]
            acc_ref[h, i] = gemm_block(h, i, 1)[0]
            rd1[h][1] = rcopy(acc_ref.at[h, i], comm_ref.at[h, 1],
                              h, 1, part1[h])

        keep = (2 * b1, 2 * hi)
        for h in range(2):
            acc_ref[h, pl.ds(keep[h], 2)] = gemm_block(h, keep[h], 2)

        rd2 = []
        for h in range(2):
            rd1[h][0].wait()
            acc_ref[h, fwd[h]] = acc_ref[h, fwd[h]] + comm_ref[h, 0]
            rd2.append(rcopy(acc_ref.at[h, fwd[h]], comm_ref.at[h, 2],
                             h, 2, part2[h]))

        for h in range(2):
            rd1[h][1].wait()
            acc_ref[h, own[h]] = acc_ref[h, own[h]] + comm_ref[h, 1]

        ag1, ag2a = [], []
        for h in range(2):
            rd2[h].wait()
            blk = jnp.maximum(acc_ref[h, own[h]] + comm_ref[h, 2], 0.0)
            acc_ref[h, own[h]] = blk
            ag1.append(rcopy(acc_ref.at[h, own[h]], comm_ref.at[h, 3],
                             h, 3, part2[h]))
            ag2a.append(rcopy(
                acc_ref.at[h, own[h]],
                out_ref.at[pl.ds(own[h] * ch, ch), pl.ds(h * cw, cw)],
                h, 4, part1[h]))
            out_ref[pl.ds(own[h] * ch, ch), h * cw:(h + 1) * cw] = blk

        ag2b = []
        for h in range(2):
            ag1[h].wait()
            out_ref[pl.ds(fwd[h] * ch, ch), h * cw:(h + 1) * cw] = comm_ref[h, 3]
            ag2b.append(rcopy(
                comm_ref.at[h, 3],
                out_ref.at[pl.ds(fwd[h] * ch, ch), pl.ds(h * cw, cw)],
                h, 5, part1[h]))

        for h in range(2):
            ag2a[h].wait()
            ag2b[h].wait()

    return pl.pallas_call(
        body,
        out_shape=jax.ShapeDtypeStruct((m, n), jnp.float32),
        in_specs=[
            pl.BlockSpec(memory_space=pltpu.VMEM),
            pl.BlockSpec(memory_space=pltpu.VMEM),
        ],
        out_specs=pl.BlockSpec(memory_space=pltpu.VMEM),
        scratch_shapes=[
            pltpu.VMEM((2, N_DEV, ch, cw), jnp.float32),
            pltpu.VMEM((2, 4, ch, cw), jnp.float32),
            pltpu.SemaphoreType.DMA((2, 6)),
            pltpu.SemaphoreType.DMA((2, 6)),
        ],
        compiler_params=pltpu.CompilerParams(collective_id=0),
    )(x, w_mat)


# device time: 46559 ns/iter; 1.0116x vs baseline; 1.0116x over previous
import jax
import jax.numpy as jnp
from jax import lax
from jax.experimental import pallas as pl
from jax.experimental.pallas import tpu as pltpu

N_DEV = 4


def kernel(x, w_mat):
    m, k_local = x.shape
    _, n = w_mat.shape
    ch = m // N_DEV
    cw = n // 2

    def body(x_ref, w_ref, out_ref, acc_ref, comm_ref, send_sems, recv_sems):
        my = lax.axis_index("i")
        bit0 = lax.rem(my, 2)
        hi = my // 2
        p_a = my + 1 - 2 * bit0
        p_b = 3 - my
        b1 = lax.rem(bit0 + hi, 2)

        barrier_sem = pltpu.get_barrier_semaphore()
        for nbr in (p_a, p_b):
            pl.semaphore_signal(
                barrier_sem, inc=1,
                device_id=(nbr,), device_id_type=pl.DeviceIdType.MESH,
            )

        send1 = (2 * (1 - b1), 2 * (1 - hi))
        own = (2 * b1 + hi, 2 * hi + bit0)
        fwd = (2 * b1 + (1 - hi), 2 * hi + (1 - bit0))
        o_first = (1 - hi, bit0)
        o_second = (hi, 1 - bit0)
        part1 = (p_a, p_b)
        part2 = (p_b, p_a)

        def rcopy(src, dst, h, sem_slot, dev):
            rdma = pltpu.make_async_remote_copy(
                src_ref=src, dst_ref=dst,
                send_sem=send_sems.at[h, sem_slot],
                recv_sem=recv_sems.at[h, sem_slot],
                device_id=(dev,), device_id_type=pl.DeviceIdType.MESH,
            )
            rdma.start()
            return rdma

        def gemm_block(h, blk_idx, nblk):
            return jnp.dot(
                x_ref[pl.ds(blk_idx * ch, nblk * ch), :],
                w_ref[:, h * cw:(h + 1) * cw],
                preferred_element_type=jnp.float32,
            ).reshape(nblk, ch, cw)

        rd1 = [[None, None], [None, None]]
        first = [send1[h] + o_first[h] for h in range(2)]
        for h in range(2):
            acc_ref[h, first[h]] = gemm_block(h, first[h], 1)[0]
        pl.semaphore_wait(barrier_sem, 2)
        for h in range(2):
            rd1[h][0] = rcopy(acc_ref.at[h, first[h]], comm_ref.at[h, 0],
                              h, 0, part1[h])
        for h in range(2):
            i = send1[h] + o_second[h]
            acc_ref[h, i] = gemm_block(h, i, 1)[0]
            rd1[h][1] = rcopy(acc_ref.at[h, i], comm_ref.at[h, 1],
                              h, 1, part1[h])

        keep = (2 * b1, 2 * hi)
        for h in range(2):
            acc_ref[h, pl.ds(keep[h], 2)] = gemm_block(h, keep[h], 2)

        rd2 = []
        for h in range(2):
            rd1[h][0].wait_recv()
            acc_ref[h, fwd[h]] = acc_ref[h, fwd[h]] + comm_ref[h, 0]
            rd2.append(rcopy(acc_ref.at[h, fwd[h]], comm_ref.at[h, 2],
                             h, 2, part2[h]))

        for h in range(2):
            rd1[h][1].wait_recv()
            acc_ref[h, own[h]] = acc_ref[h, own[h]] + comm_ref[h, 1]

        ag1, ag2a = [], []
        for h in range(2):
            rd2[h].wait_recv()
            blk = jnp.maximum(acc_ref[h, own[h]] + comm_ref[h, 2], 0.0)
            acc_ref[h, own[h]] = blk
            ag1.append(rcopy(acc_ref.at[h, own[h]], comm_ref.at[h, 3],
                             h, 3, part2[h]))
            ag2a.append(rcopy(
                acc_ref.at[h, own[h]],
                out_ref.at[pl.ds(own[h] * ch, ch), pl.ds(h * cw, cw)],
                h, 4, part1[h]))
            out_ref[pl.ds(own[h] * ch, ch), h * cw:(h + 1) * cw] = blk

        ag2b = []
        for h in range(2):
            ag1[h].wait_recv()
            out_ref[pl.ds(fwd[h] * ch, ch), h * cw:(h + 1) * cw] = comm_ref[h, 3]
            ag2b.append(rcopy(
                comm_ref.at[h, 3],
                out_ref.at[pl.ds(fwd[h] * ch, ch), pl.ds(h * cw, cw)],
                h, 5, part1[h]))

        for h in range(2):
            ag2a[h].wait_recv()
            ag2b[h].wait_recv()
        for h in range(2):
            for rdma in (rd1[h][0], rd1[h][1], rd2[h], ag1[h], ag2a[h], ag2b[h]):
                rdma.wait_send()

    return pl.pallas_call(
        body,
        out_shape=jax.ShapeDtypeStruct((m, n), jnp.float32),
        in_specs=[
            pl.BlockSpec(memory_space=pltpu.VMEM),
            pl.BlockSpec(memory_space=pltpu.VMEM),
        ],
        out_specs=pl.BlockSpec(memory_space=pltpu.VMEM),
        scratch_shapes=[
            pltpu.VMEM((2, N_DEV, ch, cw), jnp.float32),
            pltpu.VMEM((2, 4, ch, cw), jnp.float32),
            pltpu.SemaphoreType.DMA((2, 6)),
            pltpu.SemaphoreType.DMA((2, 6)),
        ],
        compiler_params=pltpu.CompilerParams(collective_id=0),
    )(x, w_mat)


# device time: 29740 ns/iter; 1.5837x vs baseline; 1.5655x over previous
import jax
import jax.numpy as jnp
from jax import lax
from jax.experimental import pallas as pl
from jax.experimental.pallas import tpu as pltpu

N_DEV = 4
BF16 = jnp.bfloat16


def kernel(x, w_mat):
    m, k_local = x.shape
    _, n = w_mat.shape
    ch = m // N_DEV
    cw = n // 2

    def body(x_ref, w_ref, out_ref, acc_ref, comm_ref, sbuf_ref,
             send_sems, recv_sems):
        my = lax.axis_index("i")
        bit0 = lax.rem(my, 2)
        hi = my // 2
        p_a = my + 1 - 2 * bit0
        p_b = 3 - my
        b1 = lax.rem(bit0 + hi, 2)

        barrier_sem = pltpu.get_barrier_semaphore()
        for nbr in (p_a, p_b):
            pl.semaphore_signal(
                barrier_sem, inc=1,
                device_id=(nbr,), device_id_type=pl.DeviceIdType.MESH,
            )

        send1 = (2 * (1 - b1), 2 * (1 - hi))
        own = (2 * b1 + hi, 2 * hi + bit0)
        fwd = (2 * b1 + (1 - hi), 2 * hi + (1 - bit0))
        o_first = (1 - hi, bit0)
        o_second = (hi, 1 - bit0)
        keep = (2 * b1, 2 * hi)
        part1 = (p_a, p_b)
        part2 = (p_b, p_a)

        def rcopy(src, dst, h, sem_slot, dev):
            rdma = pltpu.make_async_remote_copy(
                src_ref=src, dst_ref=dst,
                send_sem=send_sems.at[h, sem_slot],
                recv_sem=recv_sems.at[h, sem_slot],
                device_id=(dev,), device_id_type=pl.DeviceIdType.MESH,
            )
            rdma.start()
            return rdma

        def gemm_block(h, blk_idx, nblk):
            return jnp.dot(
                x_ref[pl.ds(blk_idx * ch, nblk * ch), :],
                w_ref[:, h * cw:(h + 1) * cw],
                preferred_element_type=jnp.float32,
            ).reshape(nblk, ch, cw)

        rd1 = [[None, None], [None, None]]
        for h in range(2):
            sbuf_ref[h, 0] = gemm_block(h, send1[h] + o_first[h], 1)[0].astype(BF16)
        pl.semaphore_wait(barrier_sem, 2)
        for h in range(2):
            rd1[h][0] = rcopy(sbuf_ref.at[h, 0], comm_ref.at[h, 0],
                              h, 0, part1[h])
        for h in range(2):
            sbuf_ref[h, 1] = gemm_block(h, send1[h] + o_second[h], 1)[0].astype(BF16)
            rd1[h][1] = rcopy(sbuf_ref.at[h, 1], comm_ref.at[h, 1],
                              h, 1, part1[h])

        for h in range(2):
            acc_ref[h, pl.ds(keep[h], 2)] = gemm_block(h, keep[h], 2)

        rd2 = []
        for h in range(2):
            rd1[h][0].wait_recv()
            sbuf_ref[h, 2] = (
                acc_ref[h, fwd[h]] + comm_ref[h, 0].astype(jnp.float32)
            ).astype(BF16)
            rd2.append(rcopy(sbuf_ref.at[h, 2], comm_ref.at[h, 2],
                             h, 2, part2[h]))

        for h in range(2):
            rd1[h][1].wait_recv()
            acc_ref[h, own[h]] = (
                acc_ref[h, own[h]] + comm_ref[h, 1].astype(jnp.float32)
            )

        ag1, ag2a = [], []
        for h in range(2):
            rd2[h].wait_recv()
            blk = jnp.maximum(
                acc_ref[h, own[h]] + comm_ref[h, 2].astype(jnp.float32), 0.0
            )
            sbuf_ref[h, 3] = blk.astype(BF16)
            ag1.append(rcopy(sbuf_ref.at[h, 3], comm_ref.at[h, 3],
                             h, 3, part2[h]))
            ag2a.append(rcopy(sbuf_ref.at[h, 3], comm_ref.at[h, 4],
                              h, 4, part1[h]))
            out_ref[pl.ds(own[h] * ch, ch), h * cw:(h + 1) * cw] = blk

        ag2b = []
        for h in range(2):
            ag1[h].wait_recv()
            out_ref[pl.ds(fwd[h] * ch, ch), h * cw:(h + 1) * cw] = (
                comm_ref[h, 3].astype(jnp.float32)
            )
            ag2b.append(rcopy(comm_ref.at[h, 3], comm_ref.at[h, 5],
                              h, 5, part1[h]))

        for h in range(2):
            ag2a[h].wait_recv()
            out_ref[pl.ds((send1[h] + o_second[h]) * ch, ch),
                    h * cw:(h + 1) * cw] = comm_ref[h, 4].astype(jnp.float32)
        for h in range(2):
            ag2b[h].wait_recv()
            out_ref[pl.ds((send1[h] + o_first[h]) * ch, ch),
                    h * cw:(h + 1) * cw] = comm_ref[h, 5].astype(jnp.float32)

        for h in range(2):
            for rdma in (rd1[h][0], rd1[h][1], rd2[h], ag1[h], ag2a[h], ag2b[h]):
                rdma.wait_send()

    return pl.pallas_call(
        body,
        out_shape=jax.ShapeDtypeStruct((m, n), jnp.float32),
        in_specs=[
            pl.BlockSpec(memory_space=pltpu.VMEM),
            pl.BlockSpec(memory_space=pltpu.VMEM),
        ],
        out_specs=pl.BlockSpec(memory_space=pltpu.VMEM),
        scratch_shapes=[
            pltpu.VMEM((2, N_DEV, ch, cw), jnp.float32),
            pltpu.VMEM((2, 6, ch, cw), BF16),
            pltpu.VMEM((2, 4, ch, cw), BF16),
            pltpu.SemaphoreType.DMA((2, 6)),
            pltpu.SemaphoreType.DMA((2, 6)),
        ],
        compiler_params=pltpu.CompilerParams(collective_id=0),
    )(x, w_mat)
